# Initial kernel scaffold; baseline (speedup 1.0000x reference)
import jax
import jax.numpy as jnp
from jax.experimental import pallas as pl


def kernel(hidden_states, gate_weight, expert_biases):
    raise NotImplementedError("write your pallas kernel here")


# speedup vs baseline: 7.4566x; 7.4566x over previous
import jax
import jax.numpy as jnp
from jax.experimental import pallas as pl
from jax.experimental.pallas import tpu as pltpu

_BV = 3200


def _moe_bias_body(hs_ref, gw_ref, eb_ref, out_ref, aux_ref, w_ref):
    @pl.when(pl.program_id(0) == 0)
    def _gate():
        logits = jax.lax.dot_general(
            hs_ref[...], gw_ref[...], (((1,), (1,)), ((), ())),
            preferred_element_type=jnp.float32)
        m = jnp.max(logits, axis=-1, keepdims=True)
        e = jnp.exp(logits - m)
        probs = e / jnp.sum(e, axis=-1, keepdims=True)
        ne = probs.shape[-1]
        eidx = jax.lax.broadcasted_iota(jnp.int32, probs.shape, 1)
        m1 = jnp.max(probs, axis=-1, keepdims=True)
        i1 = jnp.min(jnp.where(probs == m1, eidx, ne), axis=-1, keepdims=True)
        masked = jnp.where(eidx == i1, -jnp.inf, probs)
        m2 = jnp.max(masked, axis=-1, keepdims=True)
        i2 = jnp.min(jnp.where(masked == m2, eidx, ne), axis=-1, keepdims=True)
        denom = m1 + m2
        w_ref[...] = (jnp.where(eidx == i1, m1 / denom, 0.0)
                      + jnp.where(eidx == i2, m2 / denom, 0.0))
        usage = jnp.mean(probs, axis=0, keepdims=True)
        aux_ref[...] = jnp.sum(usage * jnp.log(usage), axis=-1,
                               keepdims=True) * ne

    out_ref[...] = jnp.dot(w_ref[...], eb_ref[...],
                           preferred_element_type=jnp.float32)


def kernel(hidden_states, gate_weight, expert_biases):
    t, h = hidden_states.shape
    e, v = expert_biases.shape
    bias, aux = pl.pallas_call(
        _moe_bias_body,
        grid=(v // _BV,),
        in_specs=[
            pl.BlockSpec((t, h), lambda i: (0, 0)),
            pl.BlockSpec((e, h), lambda i: (0, 0)),
            pl.BlockSpec((e, _BV), lambda i: (0, i)),
        ],
        out_specs=[
            pl.BlockSpec((t, _BV), lambda i: (0, i)),
            pl.BlockSpec((1, 1), lambda i: (0, 0)),
        ],
        out_shape=[
            jax.ShapeDtypeStruct((t, v), jnp.float32),
            jax.ShapeDtypeStruct((1, 1), jnp.float32),
        ],
        scratch_shapes=[pltpu.VMEM((t, e), jnp.float32)],
    )(hidden_states, gate_weight, expert_biases)
    return bias, aux[0, 0]
